# baseline (speedup 1.0000x reference)
import functools

import jax
import jax.numpy as jnp
from jax import lax
from jax.experimental import pallas as pl
from jax.experimental.pallas import tpu as pltpu
from jax.experimental.pallas import tpu_sc as plsc

BATCH = 16384
D = 64
NC = 2
NS = 16
NW = NC * NS
BPW = BATCH // NW
CHUNK = 128
NCHUNK = BPW // CHUNK

_mesh = plsc.VectorSubcoreMesh(core_axis_name="c", subcore_axis_name="s")


@functools.partial(
    pl.kernel,
    mesh=_mesh,
    out_type=(
        jax.ShapeDtypeStruct((BATCH, D), jnp.float32),
        jax.ShapeDtypeStruct((BATCH, D), jnp.float32),
    ),
    scratch_types=[
        pltpu.VMEM((BPW,), jnp.int32),
        pltpu.VMEM((BPW,), jnp.int32),
        pltpu.VMEM((BPW, D), jnp.float32),
        pltpu.VMEM((BPW, D), jnp.float32),
        pltpu.SemaphoreType.DMA,
    ],
    compiler_params=pltpu.CompilerParams(use_tc_tiling_on_sc=False),
)
def _gather2(user_hbm, item_hbm, utab_hbm, itab_hbm, uout_hbm, iout_hbm,
             uidx_v, iidx_v, urows_v, irows_v, sem):
    wid = lax.axis_index("s") * NC + lax.axis_index("c")
    base = wid * BPW
    pltpu.sync_copy(user_hbm.at[pl.ds(base, BPW)], uidx_v)
    pltpu.sync_copy(item_hbm.at[pl.ds(base, BPW)], iidx_v)
    copies = []
    for j in range(NCHUNK):
        sl = pl.ds(j * CHUNK, CHUNK)
        copies.append(
            pltpu.make_async_copy(utab_hbm.at[uidx_v.at[sl]], urows_v.at[sl], sem))
        copies.append(
            pltpu.make_async_copy(itab_hbm.at[iidx_v.at[sl]], irows_v.at[sl], sem))
    for c in copies:
        c.start()
    for c in copies:
        c.wait()
    pltpu.sync_copy(urows_v, uout_hbm.at[pl.ds(base, BPW)])
    pltpu.sync_copy(irows_v, iout_hbm.at[pl.ds(base, BPW)])


def kernel(user, item, user_table, item_table):
    user_e, item_e = _gather2(
        user.astype(jnp.int32), item.astype(jnp.int32), user_table, item_table)
    return (user_e, item_e)


# speedup vs baseline: 1.0077x; 1.0077x over previous
import functools

import jax
import jax.numpy as jnp
from jax import lax
from jax.experimental import pallas as pl
from jax.experimental.pallas import tpu as pltpu
from jax.experimental.pallas import tpu_sc as plsc

BATCH = 16384
D = 64
NC = 2
NS = 16
NW = NC * NS
BPW = BATCH // NW
CHUNK = 128
NCHUNK = BPW // CHUNK

_mesh = plsc.VectorSubcoreMesh(core_axis_name="c", subcore_axis_name="s")


@functools.partial(
    pl.kernel,
    mesh=_mesh,
    out_type=jax.ShapeDtypeStruct((BATCH, D), jnp.float32),
    scratch_types=[
        pltpu.VMEM((BPW,), jnp.int32),
        pltpu.VMEM((BPW, D), jnp.float32),
        pltpu.SemaphoreType.DMA,
    ],
    compiler_params=pltpu.CompilerParams(use_tc_tiling_on_sc=False),
)
def _gather1(idx_hbm, tab_hbm, out_hbm, idx_v, rows_v, sem):
    wid = lax.axis_index("s") * NC + lax.axis_index("c")
    base = wid * BPW
    pltpu.sync_copy(idx_hbm.at[pl.ds(base, BPW)], idx_v)
    copies = []
    for j in range(NCHUNK):
        sl = pl.ds(j * CHUNK, CHUNK)
        copies.append(
            pltpu.make_async_copy(tab_hbm.at[idx_v.at[sl]], rows_v.at[sl], sem))
    for c in copies:
        c.start()
    for c in copies:
        c.wait()
    pltpu.sync_copy(rows_v, out_hbm.at[pl.ds(base, BPW)])


def kernel(user, item, user_table, item_table):
    user_e = _gather1(user.astype(jnp.int32), user_table)
    item_e = _gather1(item.astype(jnp.int32), item_table)
    return (user_e, item_e)


# speedup vs baseline: 3.3147x; 3.2893x over previous
import functools

import jax
import jax.numpy as jnp
from jax import lax
from jax.experimental import pallas as pl
from jax.experimental.pallas import tpu as pltpu
from jax.experimental.pallas import tpu_sc as plsc

VOCAB = 1000001
D = 64
BATCH = 16384
NC = 2
NS = 16
NW = NC * NS
RANGE = 31360
CC = 512
NCH = RANGE // CC
C0MAX = 999424
TAIL0 = C0MAX + CC
TAILW = VOCAB - TAIL0
HITCAP = 64
OUTPAD = 128
OUTN = BATCH * D + OUTPAD

_i32 = jnp.int32
_mesh = plsc.VectorSubcoreMesh(core_axis_name="c", subcore_axis_name="s")


@functools.partial(
    pl.kernel,
    mesh=_mesh,
    out_type=(
        jax.ShapeDtypeStruct((OUTN,), jnp.float32),
        jax.ShapeDtypeStruct((OUTN,), jnp.float32),
    ),
    scratch_types=[
        pltpu.VMEM((4096,), _i32),
        pltpu.VMEM((16384,), _i32),
        pltpu.VMEM((16400,), _i32),
        pltpu.VMEM((D, CC), jnp.float32),
        pltpu.VMEM((D, CC), jnp.float32),
        pltpu.VMEM((HITCAP * D // 128, 128), jnp.float32),
        pltpu.VMEM((HITCAP * D // 128, 128), _i32),
        pltpu.VMEM((HITCAP * D // 128, 128), jnp.float32),
        pltpu.VMEM((HITCAP * D // 128, 128), _i32),
        pltpu.SemaphoreType.DMA,
        pltpu.SemaphoreType.DMA,
        pltpu.SemaphoreType.DMA,
        pltpu.SemaphoreType.DMA,
        pltpu.SemaphoreType.DMA,
    ],
    compiler_params=pltpu.CompilerParams(needs_layout_passes=False),
)
def _mf(user_hbm, item_hbm, utab_hbm, itab_hbm, utail_hbm, itail_hbm,
        uout_hbm, iout_hbm,
        idxbuf, hitk, wl, slabA, slabB, datA, adrA, datB, adrB,
        semA, semB, semSA, semSB, semM):
    wid = lax.axis_index("s") * NC + lax.axis_index("c")
    t_lo = wid * RANGE
    lanes = lax.iota(_i32, 16)
    dconsts = [lanes + 16 * g4 for g4 in range(4)]

    def fire_slab(c0, slab, sem):
        for r in range(8):
            pltpu.make_async_copy(
                _fire_src(r, c0), slab.at[pl.ds(r * 8, 8), :], sem).start()

    def wait_slab(c0, slab, sem):
        for r in range(8):
            pltpu.make_async_copy(
                _fire_src(r, c0), slab.at[pl.ds(r * 8, 8), :], sem).wait()

    for idx_hbm, tab_hbm, tail_hbm, out_hbm in (
        (user_hbm, utab_hbm, utail_hbm, uout_hbm),
        (item_hbm, itab_hbm, itail_hbm, iout_hbm),
    ):
        def _fire_src(r, c0, tab=tab_hbm):
            return tab.at[pl.ds(r * 8, 8), pl.ds(c0, CC)]

        def piece(p, cnt):
            pltpu.sync_copy(idx_hbm.at[pl.ds(p * 4096, 4096)], idxbuf)

            def group(g, cnt):
                v = idxbuf[pl.ds(g * 16, 16)]
                m = (v >= t_lo) & (v < t_lo + RANGE)
                key = (v << 10) + (p * 256 + g)
                pos = (lanes << 10) + cnt
                plsc.store_scatter(hitk, [pos], key, mask=m)
                return cnt + jnp.where(m, 1, 0).astype(_i32)

            return lax.fori_loop(0, 256, group, cnt)

        cnt = lax.fori_loop(0, 4, piece, jnp.zeros((16,), _i32))
        maxcnt = jnp.max(cnt)

        def select(c0, w):
            def scan(i, nsel):
                key = plsc.load_gather(hitk, [(lanes << 10) + i])
                v = key >> 10
                m = (i < cnt) & (v >= c0) & (v < c0 + w)
                b = ((key & 1023) << 4) + lanes
                key2 = ((v - c0) << 14) + b
                plsc.store_compressed(wl.at[pl.ds(nsel, 16)], key2, mask=m)
                ps = jnp.max(plsc.all_reduce_population_count(m))
                return nsel + ps

            return lax.fori_loop(0, maxcnt, scan, jnp.asarray(0, _i32))

        def extract(slab, dat, adr, wl_base, nbatch):
            def hit(j, carry):
                key2 = wl[pl.ds(wl_base + j, 16)][0]
                vrel = key2 >> 14
                b = key2 & 16383
                vv = jnp.broadcast_to(vrel, (16,)).astype(_i32)
                for g4 in range(4):
                    col = plsc.load_gather(slab, [dconsts[g4], vv])
                    pos = j * 64 + g4 * 16
                    r = pos >> 7
                    o = pos & 127
                    dat.at[r][pl.ds(o, 16)] = col
                    adr.at[r][pl.ds(o, 16)] = (b << 6) + g4 * 16 + lanes
                pltpu.sync_copy(
                    dat.at[j >> 1].at[pl.ds((j & 1) * 64, 64)],
                    out_hbm.at[pl.ds(b * 64, 64)])
                return carry

            lax.fori_loop(0, nbatch, hit, 0)

        def fire_scatter(dat, adr, nrows, sem, out=out_hbm):
            def row(rr, carry):
                pltpu.make_async_copy(dat.at[rr], out.at[adr.at[rr]], sem).start()
                return carry
            lax.fori_loop(0, nrows, row, 0)

        def drain_scatter(dat, adr, nrows, sem, out=out_hbm):
            def row(rr, carry):
                pltpu.make_async_copy(dat.at[rr], out.at[adr.at[rr]], sem).wait()
                return carry
            lax.fori_loop(0, nrows, row, 0)

        def process(slab, dat, adr, sem, nsel, prev_rows):
            nbat = (nsel + HITCAP - 1) // HITCAP

            def batch(bb, carry):
                pending = carry
                nb = jnp.minimum(nsel - bb * HITCAP, HITCAP)
                extract(slab, dat, adr, bb * HITCAP, nb)
                return pending

            return lax.fori_loop(0, nbat, batch, prev_rows)

        def pad_adr(adr):
            def prow(rr, carry):
                adr.at[rr][pl.ds(0, 16)] = BATCH * D + lanes
                for o in range(16, 128, 16):
                    adr.at[rr][pl.ds(o, 16)] = BATCH * D + lanes + o
                return carry
            lax.fori_loop(0, HITCAP * D // 128, prow, 0)

        pad_adr(adrA)
        pad_adr(adrB)

        def c0_of(k):
            return jnp.minimum(t_lo + k * CC, C0MAX)

        fire_slab(c0_of(0), slabA, semA)

        def pair(kk, carry):
            pendA, pendB = carry
            kA = kk * 2
            kB = kA + 1
            fire_slab(c0_of(kB), slabB, semB)
            wait_slab(c0_of(kA), slabA, semA)
            nselA = select(c0_of(kA), CC)
            pendA = process(slabA, datA, adrA, semSA, nselA, pendA)
            fire_slab(c0_of(kA + 2), slabA, semA)
            wait_slab(c0_of(kB), slabB, semB)
            nselB = select(c0_of(kB), CC)
            pendB = process(slabB, datB, adrB, semSB, nselB, pendB)
            return (pendA, pendB)

        pendA, pendB = lax.fori_loop(
            0, NCH // 2 + 1, pair,
            (jnp.asarray(0, _i32), jnp.asarray(0, _i32)))
        wait_slab(c0_of(NCH + 2), slabA, semA)
        drain_scatter(datA, adrA, pendA, semSA)
        drain_scatter(datB, adrB, pendB, semSB)

        @pl.when(wid == NW - 1)
        def _tail():
            for r in range(8):
                pltpu.sync_copy(
                    tail_hbm.at[pl.ds(r * 8, 8), :],
                    slabA.at[pl.ds(r * 8, 8), pl.ds(0, 128)])
            nsel = select(TAIL0, 128)
            pend = process(slabA, datA, adrA, semSA, nsel,
                           jnp.asarray(0, _i32))
            drain_scatter(datA, adrA, pend, semSA)


def kernel(user, item, user_table, item_table):
    utail = jnp.pad(user_table[TAIL0:], ((0, 128 - TAILW), (0, 0))).T
    itail = jnp.pad(item_table[TAIL0:], ((0, 128 - TAILW), (0, 0))).T
    u1d, i1d = _mf(
        user.astype(jnp.int32), item.astype(jnp.int32),
        user_table.T, item_table.T, utail, itail)
    user_e = u1d[:BATCH * D].reshape(BATCH, D)
    item_e = i1d[:BATCH * D].reshape(BATCH, D)
    return (user_e, item_e)


# speedup vs baseline: 3.7696x; 1.1372x over previous
import functools

import jax
import jax.numpy as jnp
from jax import lax
from jax.experimental import pallas as pl
from jax.experimental.pallas import tpu as pltpu
from jax.experimental.pallas import tpu_sc as plsc

VOCAB = 1000001
D = 64
BATCH = 16384
NC = 2
NS = 16
NW = NC * NS
RANGE = 31360
CC = 512
NCH = RANGE // CC
C0MAX = 999424
TAIL0 = C0MAX + CC
TAILW = VOCAB - TAIL0
HITCAP = 64
OUTPAD = 128
OUTN = BATCH * D + OUTPAD

_i32 = jnp.int32
_mesh = plsc.VectorSubcoreMesh(core_axis_name="c", subcore_axis_name="s")


@functools.partial(
    pl.kernel,
    mesh=_mesh,
    out_type=(
        jax.ShapeDtypeStruct((OUTN,), jnp.float32),
        jax.ShapeDtypeStruct((OUTN,), jnp.float32),
    ),
    scratch_types=[
        pltpu.VMEM((4096,), _i32),
        pltpu.VMEM((16384,), _i32),
        pltpu.VMEM((16400,), _i32),
        pltpu.VMEM((D, CC), jnp.float32),
        pltpu.VMEM((D, CC), jnp.float32),
        pltpu.VMEM((HITCAP * D // 128, 128), jnp.float32),
        pltpu.VMEM((HITCAP * D // 128, 128), _i32),
        pltpu.VMEM((HITCAP * D // 128, 128), jnp.float32),
        pltpu.VMEM((HITCAP * D // 128, 128), _i32),
        pltpu.SemaphoreType.DMA,
        pltpu.SemaphoreType.DMA,
        pltpu.SemaphoreType.DMA,
        pltpu.SemaphoreType.DMA,
        pltpu.SemaphoreType.DMA,
    ],
    compiler_params=pltpu.CompilerParams(needs_layout_passes=False),
)
def _mf(user_hbm, item_hbm, utab_hbm, itab_hbm, utail_hbm, itail_hbm,
        uout_hbm, iout_hbm,
        idxbuf, hitk, wl, slabA, slabB, datA, adrA, datB, adrB,
        semA, semB, semSA, semSB, semM):
    wid = lax.axis_index("s") * NC + lax.axis_index("c")
    t_lo = wid * RANGE
    lanes = lax.iota(_i32, 16)
    dconsts = [lanes + 16 * g4 for g4 in range(4)]

    def fire_slab(c0, slab, sem):
        for r in range(8):
            pltpu.make_async_copy(
                _fire_src(r, c0), slab.at[pl.ds(r * 8, 8), :], sem).start()

    def wait_slab(c0, slab, sem):
        for r in range(8):
            pltpu.make_async_copy(
                _fire_src(r, c0), slab.at[pl.ds(r * 8, 8), :], sem).wait()

    for idx_hbm, tab_hbm, tail_hbm, out_hbm in (
        (user_hbm, utab_hbm, utail_hbm, uout_hbm),
        (item_hbm, itab_hbm, itail_hbm, iout_hbm),
    ):
        def _fire_src(r, c0, tab=tab_hbm):
            return tab.at[pl.ds(r * 8, 8), pl.ds(c0, CC)]

        def piece(p, cnt):
            pltpu.sync_copy(idx_hbm.at[pl.ds(p * 4096, 4096)], idxbuf)

            def group(g, cnt):
                v = idxbuf[pl.ds(g * 16, 16)]
                m = (v >= t_lo) & (v < t_lo + RANGE)
                key = (v << 10) + (p * 256 + g)
                pos = (lanes << 10) + cnt
                plsc.store_scatter(hitk, [pos], key, mask=m)
                return cnt + jnp.where(m, 1, 0).astype(_i32)

            return lax.fori_loop(0, 256, group, cnt)

        cnt = lax.fori_loop(0, 4, piece, jnp.zeros((16,), _i32))
        maxcnt = jnp.max(cnt)

        def select(c0, w):
            def scan(i, nsel):
                key = plsc.load_gather(hitk, [(lanes << 10) + i])
                v = key >> 10
                m = (i < cnt) & (v >= c0) & (v < c0 + w)
                b = ((key & 1023) << 4) + lanes
                key2 = ((v - c0) << 14) + b
                plsc.store_compressed(wl.at[pl.ds(nsel, 16)], key2, mask=m)
                ps = jnp.max(plsc.all_reduce_population_count(m))
                return nsel + ps

            return lax.fori_loop(0, maxcnt, scan, jnp.asarray(0, _i32))

        def extract(slab, dat, adr, sem, wl_base, nbatch):
            def hit(j, carry):
                key2 = wl[pl.ds(wl_base + j, 16)][0]
                vrel = key2 >> 14
                b = key2 & 16383
                vv = jnp.broadcast_to(vrel, (16,)).astype(_i32)
                for g4 in range(4):
                    col = plsc.load_gather(slab, [dconsts[g4], vv])
                    pos = j * 64 + g4 * 16
                    r = pos >> 7
                    o = pos & 127
                    dat.at[r][pl.ds(o, 16)] = col
                    adr.at[r][pl.ds(o, 16)] = (b << 6) + g4 * 16 + lanes
                pltpu.make_async_copy(
                    dat.at[j >> 1].at[pl.ds((j & 1) * 64, 64)],
                    out_hbm.at[pl.ds(b * 64, 64)], sem).start()
                return carry

            lax.fori_loop(0, nbatch, hit, 0)

        def drain_hits(dat, adr, pend, sem, out=out_hbm):
            def row(j, carry):
                b = adr.at[j >> 1][pl.ds((j & 1) * 64, 16)][0] >> 6
                pltpu.make_async_copy(
                    dat.at[j >> 1].at[pl.ds((j & 1) * 64, 64)],
                    out.at[pl.ds(b * 64, 64)], sem).wait()
                return carry
            lax.fori_loop(0, pend, row, 0)

        def process(slab, dat, adr, sem, nsel, prev_pend):
            nbat = (nsel + HITCAP - 1) // HITCAP

            def batch(bb, pend):
                drain_hits(dat, adr, pend, sem)
                nb = jnp.minimum(nsel - bb * HITCAP, HITCAP)
                extract(slab, dat, adr, sem, bb * HITCAP, nb)
                return nb

            return lax.fori_loop(0, nbat, batch, prev_pend)

        def c0_of(k):
            return jnp.minimum(t_lo + k * CC, C0MAX)

        fire_slab(c0_of(0), slabA, semA)

        def pair(kk, carry):
            pendA, pendB = carry
            kA = kk * 2
            kB = kA + 1
            fire_slab(c0_of(kB), slabB, semB)
            wait_slab(c0_of(kA), slabA, semA)
            nselA = select(c0_of(kA), CC)
            pendA = process(slabA, datA, adrA, semSA, nselA, pendA)
            fire_slab(c0_of(kA + 2), slabA, semA)
            wait_slab(c0_of(kB), slabB, semB)
            nselB = select(c0_of(kB), CC)
            pendB = process(slabB, datB, adrB, semSB, nselB, pendB)
            return (pendA, pendB)

        pendA, pendB = lax.fori_loop(
            0, NCH // 2 + 1, pair,
            (jnp.asarray(0, _i32), jnp.asarray(0, _i32)))
        wait_slab(c0_of(NCH + 2), slabA, semA)
        drain_hits(datA, adrA, pendA, semSA)
        drain_hits(datB, adrB, pendB, semSB)

        @pl.when(wid == NW - 1)
        def _tail():
            for r in range(8):
                pltpu.sync_copy(
                    tail_hbm.at[pl.ds(r * 8, 8), :],
                    slabA.at[pl.ds(r * 8, 8), pl.ds(0, 128)])
            nsel = select(TAIL0, 128)
            pend = process(slabA, datA, adrA, semSA, nsel,
                           jnp.asarray(0, _i32))
            drain_hits(datA, adrA, pend, semSA)


def kernel(user, item, user_table, item_table):
    utail = jnp.pad(user_table[TAIL0:], ((0, 128 - TAILW), (0, 0))).T
    itail = jnp.pad(item_table[TAIL0:], ((0, 128 - TAILW), (0, 0))).T
    u1d, i1d = _mf(
        user.astype(jnp.int32), item.astype(jnp.int32),
        user_table.T, item_table.T, utail, itail)
    user_e = u1d[:BATCH * D].reshape(BATCH, D)
    item_e = i1d[:BATCH * D].reshape(BATCH, D)
    return (user_e, item_e)


# speedup vs baseline: 3.7831x; 1.0036x over previous
import functools

import jax
import jax.numpy as jnp
from jax import lax
from jax.experimental import pallas as pl
from jax.experimental.pallas import tpu as pltpu
from jax.experimental.pallas import tpu_sc as plsc

VOCAB = 1000001
D = 64
BATCH = 16384
NC = 2
NS = 16
NW = NC * NS
RANGE = 31360
CC = 512
NCH = RANGE // CC
C0MAX = 999424
TAIL0 = C0MAX + CC
TAILW = VOCAB - TAIL0
HITCAP = 64
OUTN = BATCH * D

_i32 = jnp.int32
_mesh = plsc.VectorSubcoreMesh(core_axis_name="c", subcore_axis_name="s")


@functools.partial(
    pl.kernel,
    mesh=_mesh,
    out_type=(
        jax.ShapeDtypeStruct((OUTN,), jnp.float32),
        jax.ShapeDtypeStruct((OUTN,), jnp.float32),
    ),
    scratch_types=[
        pltpu.VMEM((4096,), _i32),
        pltpu.VMEM((16384,), _i32),
        pltpu.VMEM((16400,), _i32),
        pltpu.VMEM((D, CC), jnp.float32),
        pltpu.VMEM((D, CC), jnp.float32),
        pltpu.VMEM((HITCAP * D // 128, 128), jnp.float32),
        pltpu.VMEM((HITCAP * D // 128, 128), _i32),
        pltpu.VMEM((HITCAP * D // 128, 128), jnp.float32),
        pltpu.VMEM((HITCAP * D // 128, 128), _i32),
        pltpu.SemaphoreType.DMA,
        pltpu.SemaphoreType.DMA,
        pltpu.SemaphoreType.DMA,
        pltpu.SemaphoreType.DMA,
        pltpu.SemaphoreType.DMA,
    ],
    compiler_params=pltpu.CompilerParams(needs_layout_passes=False),
)
def _mf(user_hbm, item_hbm, utab_hbm, itab_hbm, utail_hbm, itail_hbm,
        uout_hbm, iout_hbm,
        idxbuf, hitk, wl, slabA, slabB, datA, adrA, datB, adrB,
        semA, semB, semSA, semSB, semM):
    wid = lax.axis_index("s") * NC + lax.axis_index("c")
    t_lo = wid * RANGE
    lanes = lax.iota(_i32, 16)
    dconsts = [lanes + 16 * g4 for g4 in range(4)]

    def fire_slab(c0, slab, sem):
        for r in range(8):
            pltpu.make_async_copy(
                _fire_src(r, c0), slab.at[pl.ds(r * 8, 8), :], sem).start()

    def wait_slab(c0, slab, sem):
        for r in range(8):
            pltpu.make_async_copy(
                _fire_src(r, c0), slab.at[pl.ds(r * 8, 8), :], sem).wait()

    for idx_hbm, tab_hbm, tail_hbm, out_hbm in (
        (user_hbm, utab_hbm, utail_hbm, uout_hbm),
        (item_hbm, itab_hbm, itail_hbm, iout_hbm),
    ):
        def _fire_src(r, c0, tab=tab_hbm):
            return tab.at[pl.ds(r * 8, 8), pl.ds(c0, CC)]

        def piece(p, cnt):
            pltpu.sync_copy(idx_hbm.at[pl.ds(p * 4096, 4096)], idxbuf)

            def group(g, cnt):
                v = idxbuf[pl.ds(g * 16, 16)]
                m = (v >= t_lo) & (v < t_lo + RANGE)
                key = (v << 10) + (p * 256 + g)
                pos = (lanes << 10) + cnt
                plsc.store_scatter(hitk, [pos], key, mask=m)
                return cnt + jnp.where(m, 1, 0).astype(_i32)

            return lax.fori_loop(0, 256, group, cnt)

        cnt = lax.fori_loop(0, 4, piece, jnp.zeros((16,), _i32))
        maxcnt = jnp.max(cnt)

        def select(c0, w):
            def scan(i, nsel):
                key = plsc.load_gather(hitk, [(lanes << 10) + i])
                v = key >> 10
                m = (i < cnt) & (v >= c0) & (v < c0 + w)
                b = ((key & 1023) << 4) + lanes
                key2 = ((v - c0) << 14) + b
                plsc.store_compressed(wl.at[pl.ds(nsel, 16)], key2, mask=m)
                ps = jnp.max(plsc.all_reduce_population_count(m))
                return nsel + ps

            return lax.fori_loop(0, maxcnt, scan, jnp.asarray(0, _i32))

        def extract(slab, dat, adr, sem, wl_base, nbatch):
            def hit(j, carry):
                key2 = wl[pl.ds(wl_base + j, 16)][0]
                vrel = key2 >> 14
                b = key2 & 16383
                vv = jnp.broadcast_to(vrel, (16,)).astype(_i32)
                for g4 in range(4):
                    col = plsc.load_gather(slab, [dconsts[g4], vv])
                    pos = j * 64 + g4 * 16
                    r = pos >> 7
                    o = pos & 127
                    dat.at[r][pl.ds(o, 16)] = col
                    adr.at[r][pl.ds(o, 16)] = (b << 6) + g4 * 16 + lanes
                pltpu.make_async_copy(
                    dat.at[j >> 1].at[pl.ds((j & 1) * 64, 64)],
                    out_hbm.at[pl.ds(b * 64, 64)], sem).start()
                return carry

            lax.fori_loop(0, nbatch, hit, 0)

        def drain_hits(dat, adr, pend, sem, out=out_hbm):
            def row(j, carry):
                b = adr.at[j >> 1][pl.ds((j & 1) * 64, 16)][0] >> 6
                pltpu.make_async_copy(
                    dat.at[j >> 1].at[pl.ds((j & 1) * 64, 64)],
                    out.at[pl.ds(b * 64, 64)], sem).wait()
                return carry
            lax.fori_loop(0, pend, row, 0)

        def process(slab, dat, adr, sem, nsel, prev_pend):
            nbat = (nsel + HITCAP - 1) // HITCAP

            def batch(bb, pend):
                drain_hits(dat, adr, pend, sem)
                nb = jnp.minimum(nsel - bb * HITCAP, HITCAP)
                extract(slab, dat, adr, sem, bb * HITCAP, nb)
                return nb

            return lax.fori_loop(0, nbat, batch, prev_pend)

        def c0_of(k):
            return jnp.minimum(t_lo + k * CC, C0MAX)

        fire_slab(c0_of(0), slabA, semA)

        def pair(kk, carry):
            pendA, pendB = carry
            kA = kk * 2
            kB = kA + 1
            fire_slab(c0_of(kB), slabB, semB)
            wait_slab(c0_of(kA), slabA, semA)
            nselA = select(c0_of(kA), CC)
            pendA = process(slabA, datA, adrA, semSA, nselA, pendA)
            fire_slab(c0_of(kA + 2), slabA, semA)
            wait_slab(c0_of(kB), slabB, semB)
            nselB = select(c0_of(kB), CC)
            pendB = process(slabB, datB, adrB, semSB, nselB, pendB)
            return (pendA, pendB)

        pendA, pendB = lax.fori_loop(
            0, NCH // 2 + 1, pair,
            (jnp.asarray(0, _i32), jnp.asarray(0, _i32)))
        wait_slab(c0_of(NCH + 2), slabA, semA)
        drain_hits(datA, adrA, pendA, semSA)
        drain_hits(datB, adrB, pendB, semSB)

        @pl.when(wid == NW - 1)
        def _tail():
            for r in range(8):
                pltpu.sync_copy(
                    tail_hbm.at[pl.ds(r * 8, 8), :],
                    slabA.at[pl.ds(r * 8, 8), pl.ds(0, 128)])
            nsel = select(TAIL0, 128)
            pend = process(slabA, datA, adrA, semSA, nsel,
                           jnp.asarray(0, _i32))
            drain_hits(datA, adrA, pend, semSA)


def kernel(user, item, user_table, item_table):
    utail = jnp.pad(user_table[TAIL0:], ((0, 128 - TAILW), (0, 0))).T
    itail = jnp.pad(item_table[TAIL0:], ((0, 128 - TAILW), (0, 0))).T
    u1d, i1d = _mf(
        user.astype(jnp.int32), item.astype(jnp.int32),
        user_table.T, item_table.T, utail, itail)
    user_e = u1d.reshape(BATCH, D)
    item_e = i1d.reshape(BATCH, D)
    return (user_e, item_e)


# speedup vs baseline: 3.7950x; 1.0032x over previous
import functools

import jax
import jax.numpy as jnp
from jax import lax
from jax.experimental import pallas as pl
from jax.experimental.pallas import tpu as pltpu
from jax.experimental.pallas import tpu_sc as plsc

VOCAB = 1000001
D = 64
BATCH = 16384
NC = 2
NS = 16
NW = NC * NS
RANGE = 31360
CC = 512
NCH = RANGE // CC
C0MAX = 999424
TAIL0 = C0MAX + CC
TAILW = VOCAB - TAIL0
HITCAP = 64
OUTN = BATCH * D

_i32 = jnp.int32
_mesh = plsc.VectorSubcoreMesh(core_axis_name="c", subcore_axis_name="s")


@functools.partial(
    pl.kernel,
    mesh=_mesh,
    out_type=(
        jax.ShapeDtypeStruct((OUTN,), jnp.float32),
        jax.ShapeDtypeStruct((OUTN,), jnp.float32),
    ),
    scratch_types=[
        pltpu.VMEM((4096,), _i32),
        pltpu.VMEM((16384,), _i32),
        pltpu.VMEM((16400,), _i32),
        pltpu.VMEM((D, CC), jnp.float32),
        pltpu.VMEM((D, CC), jnp.float32),
        pltpu.VMEM((HITCAP * D // 128, 128), jnp.float32),
        pltpu.VMEM((HITCAP * D // 128, 128), _i32),
        pltpu.VMEM((HITCAP * D // 128, 128), jnp.float32),
        pltpu.VMEM((HITCAP * D // 128, 128), _i32),
        pltpu.SemaphoreType.DMA,
        pltpu.SemaphoreType.DMA,
        pltpu.SemaphoreType.DMA,
        pltpu.SemaphoreType.DMA,
        pltpu.SemaphoreType.DMA,
    ],
    compiler_params=pltpu.CompilerParams(needs_layout_passes=False),
)
def _mf(user_hbm, item_hbm, utab_hbm, itab_hbm, utail_hbm, itail_hbm,
        uout_hbm, iout_hbm,
        idxbuf, hitk, wl, slabA, slabB, datA, adrA, datB, adrB,
        semA, semB, semSA, semSB, semM):
    wid = lax.axis_index("s") * NC + lax.axis_index("c")
    t_lo = wid * RANGE
    lanes = lax.iota(_i32, 16)
    dconsts = [lanes + 16 * g4 for g4 in range(4)]

    def fire_slab(c0, slab, sem):
        pltpu.make_async_copy(_fire_src(c0), slab, sem).start()

    def wait_slab(c0, slab, sem):
        pltpu.make_async_copy(_fire_src(c0), slab, sem).wait()

    for idx_hbm, tab_hbm, tail_hbm, out_hbm in (
        (user_hbm, utab_hbm, utail_hbm, uout_hbm),
        (item_hbm, itab_hbm, itail_hbm, iout_hbm),
    ):
        def _fire_src(c0, tab=tab_hbm):
            return tab.at[:, pl.ds(c0, CC)]

        def piece(p, cnt):
            pltpu.sync_copy(idx_hbm.at[pl.ds(p * 4096, 4096)], idxbuf)

            def group(g, cnt):
                v = idxbuf[pl.ds(g * 16, 16)]
                m = (v >= t_lo) & (v < t_lo + RANGE)
                key = (v << 10) + (p * 256 + g)
                pos = (lanes << 10) + cnt
                plsc.store_scatter(hitk, [pos], key, mask=m)
                return cnt + jnp.where(m, 1, 0).astype(_i32)

            return lax.fori_loop(0, 256, group, cnt)

        cnt = lax.fori_loop(0, 4, piece, jnp.zeros((16,), _i32))
        maxcnt = jnp.max(cnt)

        def select(c0, w):
            def scan(i, nsel):
                key = plsc.load_gather(hitk, [(lanes << 10) + i])
                v = key >> 10
                m = (i < cnt) & (v >= c0) & (v < c0 + w)
                b = ((key & 1023) << 4) + lanes
                key2 = ((v - c0) << 14) + b
                plsc.store_compressed(wl.at[pl.ds(nsel, 16)], key2, mask=m)
                ps = jnp.max(plsc.all_reduce_population_count(m))
                return nsel + ps

            return lax.fori_loop(0, maxcnt, scan, jnp.asarray(0, _i32))

        def extract(slab, dat, adr, sem, wl_base, nbatch):
            def hit(j, carry):
                key2 = wl[pl.ds(wl_base + j, 16)][0]
                vrel = key2 >> 14
                b = key2 & 16383
                vv = jnp.broadcast_to(vrel, (16,)).astype(_i32)
                for g4 in range(4):
                    col = plsc.load_gather(slab, [dconsts[g4], vv])
                    pos = j * 64 + g4 * 16
                    r = pos >> 7
                    o = pos & 127
                    dat.at[r][pl.ds(o, 16)] = col
                    adr.at[r][pl.ds(o, 16)] = (b << 6) + g4 * 16 + lanes
                pltpu.make_async_copy(
                    dat.at[j >> 1].at[pl.ds((j & 1) * 64, 64)],
                    out_hbm.at[pl.ds(b * 64, 64)], sem).start()
                return carry

            lax.fori_loop(0, nbatch, hit, 0)

        def drain_hits(dat, adr, pend, sem, out=out_hbm):
            def row(j, carry):
                b = adr.at[j >> 1][pl.ds((j & 1) * 64, 16)][0] >> 6
                pltpu.make_async_copy(
                    dat.at[j >> 1].at[pl.ds((j & 1) * 64, 64)],
                    out.at[pl.ds(b * 64, 64)], sem).wait()
                return carry
            lax.fori_loop(0, pend, row, 0)

        def process(slab, dat, adr, sem, nsel, prev_pend):
            nbat = (nsel + HITCAP - 1) // HITCAP

            def batch(bb, pend):
                drain_hits(dat, adr, pend, sem)
                nb = jnp.minimum(nsel - bb * HITCAP, HITCAP)
                extract(slab, dat, adr, sem, bb * HITCAP, nb)
                return nb

            return lax.fori_loop(0, nbat, batch, prev_pend)

        def c0_of(k):
            return jnp.minimum(t_lo + k * CC, C0MAX)

        fire_slab(c0_of(0), slabA, semA)

        def pair(kk, carry):
            pendA, pendB = carry
            kA = kk * 2
            kB = kA + 1
            fire_slab(c0_of(kB), slabB, semB)
            wait_slab(c0_of(kA), slabA, semA)
            nselA = select(c0_of(kA), CC)
            pendA = process(slabA, datA, adrA, semSA, nselA, pendA)
            fire_slab(c0_of(kA + 2), slabA, semA)
            wait_slab(c0_of(kB), slabB, semB)
            nselB = select(c0_of(kB), CC)
            pendB = process(slabB, datB, adrB, semSB, nselB, pendB)
            return (pendA, pendB)

        pendA, pendB = lax.fori_loop(
            0, NCH // 2 + 1, pair,
            (jnp.asarray(0, _i32), jnp.asarray(0, _i32)))
        wait_slab(c0_of(NCH + 2), slabA, semA)
        drain_hits(datA, adrA, pendA, semSA)
        drain_hits(datB, adrB, pendB, semSB)

        @pl.when(wid == NW - 1)
        def _tail():
            for r in range(8):
                pltpu.sync_copy(
                    tail_hbm.at[pl.ds(r * 8, 8), :],
                    slabA.at[pl.ds(r * 8, 8), pl.ds(0, 128)])
            nsel = select(TAIL0, 128)
            pend = process(slabA, datA, adrA, semSA, nsel,
                           jnp.asarray(0, _i32))
            drain_hits(datA, adrA, pend, semSA)


def kernel(user, item, user_table, item_table):
    utail = jnp.pad(user_table[TAIL0:], ((0, 128 - TAILW), (0, 0))).T
    itail = jnp.pad(item_table[TAIL0:], ((0, 128 - TAILW), (0, 0))).T
    u1d, i1d = _mf(
        user.astype(jnp.int32), item.astype(jnp.int32),
        user_table.T, item_table.T, utail, itail)
    user_e = u1d.reshape(BATCH, D)
    item_e = i1d.reshape(BATCH, D)
    return (user_e, item_e)
